# baseline (speedup 1.0000x reference)
import functools

import jax
import jax.numpy as jnp
from jax import lax
from jax.experimental import pallas as pl
from jax.experimental.pallas import tpu as pltpu
from jax.experimental.pallas import tpu_sc as plsc

BATCH = 4096
MAXLEN = 200
EMBED = 64
LANES = 16
VGROUPS = EMBED // LANES
BCHUNKS = 128 // LANES

NUM_CORES = 2
NUM_SUBCORES = 16

SPAN = 2
NTASKS = MAXLEN // SPAN
NROWS = SPAN * 128
DEPTH = 5
PRE = 3


def _body(x_hbm, tok_hbm, pos_hbm, out_hbm,
          idx0, idx1, idx2, idx3, idx4,
          buf0, buf1, buf2, buf3, buf4, x_v, pos_v,
          gsem, osem):
    idx_v = (idx0, idx1, idx2, idx3, idx4)
    bufs = (buf0, buf1, buf2, buf3, buf4)
    wid = lax.axis_index("s") * NUM_CORES + lax.axis_index("c")

    pltpu.sync_copy(x_hbm.at[pl.ds(wid * 128, 128)], x_v)
    pltpu.sync_copy(pos_hbm, pos_v)

    iota = lax.iota(jnp.int32, LANES)
    b_ids = [iota + (c * LANES) for c in range(BCHUNKS)]

    def idx_fill(t, p):
        for j in range(SPAN):
            s_splat = jnp.full((LANES,), t * SPAN + j, dtype=jnp.int32)
            for c in range(BCHUNKS):
                v = plsc.load_gather(x_v, [b_ids[c], s_splat])
                idx_v[p][j, pl.ds(c * LANES, LANES)] = v

    def gather_start(p):
        for j in range(SPAN):
            pltpu.async_copy(
                tok_hbm.at[idx_v[p].at[j]],
                bufs[p].at[pl.ds(j * 128, 128)], gsem.at[p])

    def gather_wait(p):
        for j in range(SPAN):
            pltpu.make_async_copy(
                tok_hbm.at[idx_v[p].at[j]],
                bufs[p].at[pl.ds(j * 128, 128)], gsem.at[p]).wait()

    def wb_start(t, p):
        for j in range(SPAN):
            pltpu.async_copy(
                bufs[p].at[pl.ds(j * 128, 128)],
                out_hbm.at[pl.ds(wid * 128, 128), t * SPAN + j], osem.at[p])

    def wb_wait(p):
        for j in range(SPAN):
            pltpu.make_async_copy(
                bufs[p].at[pl.ds(j * 128, 128)],
                out_hbm.at[pl.ds(wid * 128, 128), j], osem.at[p]).wait()

    def add_pos(t, p):
        for j in range(SPAN):
            s = t * SPAN + j
            pos_regs = [pos_v[s, pl.ds(g * LANES, LANES)] for g in range(VGROUPS)]

            @plsc.parallel_loop(0, 128, unroll=8)
            def _(b):
                for g in range(VGROUPS):
                    plsc.addupdate(
                        bufs[p].at[j * 128 + b, pl.ds(g * LANES, LANES)],
                        pos_regs[g])

    for t in range(PRE):
        idx_fill(t, t)
        gather_start(t)

    def task_body(t, p, p2, prefetch, drain):
        gather_wait(p)
        if prefetch:
            idx_fill(t + PRE, p2)
        add_pos(t, p)
        wb_start(t, p)
        if prefetch:
            if drain:
                wb_wait(p2)
            gather_start(p2)

    for t in range(2):
        task_body(t, t % DEPTH, (t + PRE) % DEPTH, True, False)

    def grp_body(gg, _):
        base = 2 + gg * DEPTH
        for k in range(DEPTH):
            t = base + k
            task_body(t, (2 + k) % DEPTH, k % DEPTH, True, True)
        return 0

    lax.fori_loop(0, 19, grp_body, 0)

    for t in range(97, NTASKS):
        task_body(t, t % DEPTH, (t + PRE) % DEPTH, False, False)

    for t in range(NTASKS - DEPTH, NTASKS):
        wb_wait(t % DEPTH)


@jax.jit
def _embed(x, token_table, pos_table):
    mesh = plsc.VectorSubcoreMesh(core_axis_name="c", subcore_axis_name="s")
    k = functools.partial(
        pl.kernel,
        mesh=mesh,
        out_type=jax.ShapeDtypeStruct((BATCH, MAXLEN, EMBED), jnp.float32),
        scratch_types=[
            pltpu.VMEM((SPAN, 128), jnp.int32),
            pltpu.VMEM((SPAN, 128), jnp.int32),
            pltpu.VMEM((SPAN, 128), jnp.int32),
            pltpu.VMEM((SPAN, 128), jnp.int32),
            pltpu.VMEM((SPAN, 128), jnp.int32),
            pltpu.VMEM((NROWS, EMBED), jnp.float32),
            pltpu.VMEM((NROWS, EMBED), jnp.float32),
            pltpu.VMEM((NROWS, EMBED), jnp.float32),
            pltpu.VMEM((NROWS, EMBED), jnp.float32),
            pltpu.VMEM((NROWS, EMBED), jnp.float32),
            pltpu.VMEM((128, MAXLEN), jnp.int32),
            pltpu.VMEM((MAXLEN, EMBED), jnp.float32),
            pltpu.SemaphoreType.DMA((DEPTH,)),
            pltpu.SemaphoreType.DMA((DEPTH,)),
        ],
        compiler_params=pltpu.CompilerParams(use_tc_tiling_on_sc=False, needs_layout_passes=False),
    )(_body)
    return k(x, token_table, pos_table)


def kernel(x, token_table, pos_table):
    return _embed(x.astype(jnp.int32), token_table, pos_table)
